# baseline (speedup 1.0000x reference)
import functools

import jax
import jax.numpy as jnp
from jax import lax
from jax.experimental import pallas as pl
from jax.experimental.pallas import tpu as pltpu
from jax.experimental.pallas import tpu_sc as plsc

_NS = 16
_NW = 32
_K = 128
_NBUF = 2


@functools.lru_cache(maxsize=None)
def _make_gather(B, V, D, dtype):
    n_full = B // _K
    tail = B - n_full * _K
    m_lo = (n_full // _NW) // _NBUF * _NBUF
    rem = n_full - _NW * m_lo
    n_hi = rem // _NBUF
    m_hi = m_lo + _NBUF
    n_extra = rem - n_hi * _NBUF
    extra_start = n_hi * m_hi + (_NW - n_hi) * m_lo
    slab_hi = m_hi * _K
    slab_lo = m_lo * _K
    slab_last = slab_lo + n_extra * _K + tail
    slab_max = max(slab_hi, slab_last)

    v_chunk = (V // _NS) // 8 * 8
    v_last_off = v_chunk * (_NS - 1)
    v_last = V - v_last_off

    mesh = plsc.VectorSubcoreMesh(core_axis_name="c", subcore_axis_name="s")

    @functools.partial(
        pl.kernel,
        mesh=mesh,
        out_type=jax.ShapeDtypeStruct((B, D), dtype),
        scratch_types=[
            pltpu.VMEM_SHARED((V, D), dtype),
            pltpu.VMEM((slab_max,), jnp.int32),
            *[pltpu.VMEM((_K, D), dtype) for _ in range(_NBUF)],
            *[pltpu.SemaphoreType.DMA for _ in range(2 * _NBUF)],
        ],
    )
    def gather_kernel(table_hbm, idx_hbm, out_hbm, shared, idx_v,
                      *bufs_and_sems):
        rows = bufs_and_sems[:_NBUF]
        sg = bufs_and_sems[_NBUF:2 * _NBUF]
        sw = bufs_and_sems[2 * _NBUF:]
        c = lax.axis_index("c")
        s = lax.axis_index("s")
        wid = s * 2 + c
        m = jnp.where(wid < n_hi, m_hi, m_lo)
        start = jnp.where(wid < n_hi, wid * m_hi,
                          n_hi * m_hi + (wid - n_hi) * m_lo)
        base = start * _K

        @pl.when(wid < n_hi)
        def _stage_idx_hi():
            pltpu.sync_copy(idx_hbm.at[pl.ds(base, slab_hi)],
                            idx_v.at[pl.ds(0, slab_hi)])

        @pl.when(jnp.logical_and(wid >= n_hi, wid < _NW - 1))
        def _stage_idx_lo():
            pltpu.sync_copy(idx_hbm.at[pl.ds(base, slab_lo)],
                            idx_v.at[pl.ds(0, slab_lo)])

        @pl.when(wid == _NW - 1)
        def _stage_idx_last():
            pltpu.sync_copy(idx_hbm.at[pl.ds(base, slab_last)],
                            idx_v.at[pl.ds(0, slab_last)])

        @pl.when(s < _NS - 1)
        def _stage_main():
            pltpu.sync_copy(table_hbm.at[pl.ds(s * v_chunk, v_chunk)],
                            shared.at[pl.ds(s * v_chunk, v_chunk)])

        @pl.when(s == _NS - 1)
        def _stage_last():
            pltpu.sync_copy(table_hbm.at[pl.ds(v_last_off, v_last)],
                            shared.at[pl.ds(v_last_off, v_last)])

        def g(j, b):
            return pltpu.async_copy(
                shared.at[idx_v.at[pl.ds(j * _K, _K)]], rows[b], sg[b])

        def g_drain(j, b):
            pltpu.make_async_copy(
                shared.at[idx_v.at[pl.ds(j * _K, _K)]], rows[b],
                sg[b]).wait()

        def w(j, b):
            return pltpu.async_copy(
                rows[b], out_hbm.at[pl.ds(base + j * _K, _K)], sw[b])

        for b in range(_NBUF):
            pltpu.async_copy(table_hbm.at[idx_v.at[pl.ds(b * _K, _K)]],
                             rows[b], sg[b])
        plsc.subcore_barrier()

        def body(p, carry):
            for b in range(_NBUF):
                j = p * _NBUF + b
                g_drain(j, b)
                w(j, b).wait()
                g(j + _NBUF, b)
            return carry

        lax.fori_loop(0, m // _NBUF - 1, body, 0)

        writes = []
        for b in range(_NBUF):
            j = m - _NBUF + b
            g_drain(j, b)
            writes.append(w(j, b))
        for wr in writes:
            wr.wait()

        if n_extra or tail:
            @pl.when(wid == _NW - 1)
            def _finish():
                for t in range(n_extra):
                    off = slab_lo + t * _K
                    row0 = extra_start * _K + t * _K
                    pltpu.async_copy(
                        shared.at[idx_v.at[pl.ds(off, _K)]], rows[0],
                        sg[0]).wait()
                    pltpu.async_copy(
                        rows[0], out_hbm.at[pl.ds(row0, _K)], sw[0]).wait()
                if tail:
                    off = slab_lo + n_extra * _K
                    row0 = (extra_start + n_extra) * _K
                    pltpu.async_copy(
                        shared.at[idx_v.at[pl.ds(off, tail)]],
                        rows[0].at[pl.ds(0, tail)], sg[0]).wait()
                    pltpu.async_copy(
                        rows[0].at[pl.ds(0, tail)],
                        out_hbm.at[pl.ds(row0, tail)], sw[0]).wait()

    return gather_kernel


def kernel(x_coarse, P):
    B = P.shape[0]
    V, D = x_coarse.shape
    idx = P.astype(jnp.int32)
    return _make_gather(B, V, D, x_coarse.dtype)(x_coarse, idx)
